# baseline (speedup 1.0000x reference)
import functools

import jax
import jax.numpy as jnp
from jax import lax
from jax.experimental import pallas as pl
from jax.experimental.pallas import tpu as pltpu
from jax.experimental.pallas import tpu_sc as plsc

_B, _T, _C = 128, 196, 768
_M = 4
_S = _T // _M
_H = _B // 4
_NQ = _B // _H
_COV = _H + 8
_NVC = _C // 16


def _sc_permute_t(xt):
    info = plsc.get_sparse_core_info()
    nw = info.num_cores * info.num_subcores
    wpq = nw // _M
    nk = (_S + wpq - 1) // wpq
    mesh = plsc.VectorSubcoreMesh(core_axis_name="c", subcore_axis_name="s")

    @functools.partial(
        pl.kernel,
        out_type=jax.ShapeDtypeStruct((_T, _B, _C), jnp.float32),
        mesh=mesh,
        scratch_types=[
            pltpu.VMEM((2, _COV, _C), jnp.float32),
            pltpu.VMEM((2, _H, _C), jnp.float32),
            pltpu.SemaphoreType.DMA,
            pltpu.SemaphoreType.DMA,
        ],
    )
    def k(x_hbm, out_hbm, bufs, stage, sem_in, sem_out):
        wid = lax.axis_index("s") * info.num_cores + lax.axis_index("c")
        mm = wid // wpq
        v = lax.rem(wid, wpq)
        n = nk * _NQ

        def unit(j):
            kk = j // _NQ
            h = lax.rem(j, _NQ)
            p = lax.rem(j, 2)
            t = _S * mm + lax.min(v + wpq * kk, _S - 1)
            off = pl.multiple_of(_H * h, _H)
            woff = pl.multiple_of(lax.rem(_H * h + _H, _B), 8)
            ins = [
                pltpu.make_async_copy(
                    x_hbm.at[t, pl.ds(off, _H)],
                    bufs.at[p, pl.ds(0, _H)], sem_in),
                pltpu.make_async_copy(
                    x_hbm.at[t, pl.ds(woff, 8)],
                    bufs.at[p, pl.ds(_H, 8)], sem_in),
            ]
            out = pltpu.make_async_copy(
                stage.at[p, pl.ds(0, _H)],
                out_hbm.at[t, pl.ds(off, _H)], sem_out)
            return ins, out

        def shift(p):
            @plsc.parallel_loop(0, _H, unroll=8)
            def _(r):
                src = bufs.at[p, r + mm]
                dst = stage.at[p, r]
                for vv in range(_NVC):
                    dst[pl.ds(vv * 16, 16)] = src[pl.ds(vv * 16, 16)]

        for c in unit(0)[0]:
            c.start()

        def pipeline_step(j, carry):
            ins_j, out_j = unit(j)
            for c in ins_j:
                c.wait()

            @pl.when(j >= 1)
            def _():
                out_j.wait()

            @pl.when(j + 1 < n)
            def _():
                for c in unit(j + 1)[0]:
                    c.start()
            shift(lax.rem(j, 2))
            out_j.start()
            return carry

        lax.fori_loop(0, n, pipeline_step, 0)
        unit(n - 1)[1].wait()

    return k(xt)


def kernel(patches, m):
    del m
    xt = jnp.transpose(patches, (1, 0, 2))
    mixed = jnp.transpose(_sc_permute_t(xt), (1, 0, 2))
    ids_b = jnp.arange(_B).reshape(-1, 1)
    target = (ids_b + jnp.arange(_M)) % _B
    mix_target = (ids_b - _M + 1 + jnp.arange(_M * 2 - 1) + _B) % _B
    return (mixed, target, mix_target)


# speedup vs baseline: 1.0171x; 1.0171x over previous
import functools

import jax
import jax.numpy as jnp
from jax import lax
from jax.experimental import pallas as pl
from jax.experimental.pallas import tpu as pltpu
from jax.experimental.pallas import tpu_sc as plsc

_B, _T, _C = 128, 196, 768
_M = 4
_S = _T // _M
_H = _B // 4
_NQ = _B // _H
_COV = _H + 8
_NVC = _C // 16


def _sc_permute_t(xt):
    info = plsc.get_sparse_core_info()
    nw = info.num_cores * info.num_subcores
    wpq = nw // _M
    nk = (_S + wpq - 1) // wpq
    mesh = plsc.VectorSubcoreMesh(core_axis_name="c", subcore_axis_name="s")

    @functools.partial(
        pl.kernel,
        out_type=jax.ShapeDtypeStruct((_T, _B, _C), jnp.float32),
        mesh=mesh,
        scratch_types=[
            pltpu.VMEM((2, _COV, _C), jnp.float32),
            pltpu.VMEM((2, _H, _C), jnp.float32),
            pltpu.SemaphoreType.DMA,
            pltpu.SemaphoreType.DMA,
        ],
    )
    def k(x_hbm, out_hbm, bufs, stage, sem_in, sem_out):
        wid = lax.axis_index("s") * info.num_cores + lax.axis_index("c")
        mm = wid // wpq
        v = lax.rem(wid, wpq)
        n = nk * _NQ

        def unit(j):
            kk = j // _NQ
            h = lax.rem(j, _NQ)
            p = lax.rem(j, 2)
            t = _S * mm + lax.min(v + wpq * kk, _S - 1)
            off = pl.multiple_of(_H * h, _H)
            woff = pl.multiple_of(lax.rem(_H * h + _H, _B), 8)
            merged = pltpu.make_async_copy(
                x_hbm.at[t, pl.ds(off, _COV)],
                bufs.at[p, pl.ds(0, _COV)], sem_in)
            split = [
                pltpu.make_async_copy(
                    x_hbm.at[t, pl.ds(off, _H)],
                    bufs.at[p, pl.ds(0, _H)], sem_in),
                pltpu.make_async_copy(
                    x_hbm.at[t, pl.ds(woff, 8)],
                    bufs.at[p, pl.ds(_H, 8)], sem_in),
            ]
            out = pltpu.make_async_copy(
                stage.at[p, pl.ds(0, _H)],
                out_hbm.at[t, pl.ds(off, _H)], sem_out)
            return h, merged, split, out

        def start_ins(j):
            h, merged, split, _ = unit(j)

            @pl.when(h < _NQ - 1)
            def _():
                merged.start()

            @pl.when(h == _NQ - 1)
            def _():
                for c in split:
                    c.start()

        def shift(p):
            @plsc.parallel_loop(0, _H, unroll=4)
            def _(r):
                src = bufs.at[p, r + mm]
                dst = stage.at[p, r]
                for vv in range(_NVC):
                    dst[pl.ds(vv * 16, 16)] = src[pl.ds(vv * 16, 16)]

        start_ins(0)

        def pipeline_step(j, carry):
            _, _, split_j, out_j = unit(j)
            for c in split_j:
                c.wait()

            @pl.when(j >= 1)
            def _():
                out_j.wait()

            @pl.when(j + 1 < n)
            def _():
                start_ins(j + 1)
            shift(lax.rem(j, 2))
            out_j.start()
            return carry

        lax.fori_loop(0, n, pipeline_step, 0)
        unit(n - 1)[3].wait()

    return k(xt)


def kernel(patches, m):
    del m
    xt = jnp.transpose(patches, (1, 0, 2))
    mixed = jnp.transpose(_sc_permute_t(xt), (1, 0, 2))
    ids_b = jnp.arange(_B).reshape(-1, 1)
    target = (ids_b + jnp.arange(_M)) % _B
    mix_target = (ids_b - _M + 1 + jnp.arange(_M * 2 - 1) + _B) % _B
    return (mixed, target, mix_target)


# speedup vs baseline: 1.1462x; 1.1269x over previous
import functools

import jax
import jax.numpy as jnp
from jax import lax
from jax.experimental import pallas as pl
from jax.experimental.pallas import tpu as pltpu
from jax.experimental.pallas import tpu_sc as plsc

_B, _T, _C = 128, 196, 768
_M = 4
_S = _T // _M
_H = _B // 4
_NQ = _B // _H
_COV = _H + 8
_NVC = _C // 16


def _sc_permute_t(xt):
    info = plsc.get_sparse_core_info()
    nw = info.num_cores * info.num_subcores
    wpq = nw // _M
    mesh = plsc.VectorSubcoreMesh(core_axis_name="c", subcore_axis_name="s")

    @functools.partial(
        pl.kernel,
        out_type=jax.ShapeDtypeStruct((_T, _B, _C), jnp.float32),
        mesh=mesh,
        scratch_types=[
            pltpu.VMEM((2, _COV, _C), jnp.float32),
            pltpu.VMEM((2, _H, _C), jnp.float32),
            pltpu.SemaphoreType.DMA,
            pltpu.SemaphoreType.DMA,
        ],
    )
    def k(x_hbm, out_hbm, bufs, stage, sem_in, sem_out):
        wid = lax.axis_index("s") * info.num_cores + lax.axis_index("c")
        mm = wid // wpq
        v = lax.rem(wid, wpq)
        total = _S * _NQ
        u0 = (total * v) // wpq
        n = (total * (v + 1)) // wpq - u0

        def unit(j):
            u = u0 + j
            h = lax.rem(u, _NQ)
            p = lax.rem(j, 2)
            t = _S * mm + u // _NQ
            off = pl.multiple_of(_H * h, _H)
            woff = pl.multiple_of(lax.rem(_H * h + _H, _B), 8)
            merged = pltpu.make_async_copy(
                x_hbm.at[t, pl.ds(off, _COV)],
                bufs.at[p, pl.ds(0, _COV)], sem_in)
            split = [
                pltpu.make_async_copy(
                    x_hbm.at[t, pl.ds(off, _H)],
                    bufs.at[p, pl.ds(0, _H)], sem_in),
                pltpu.make_async_copy(
                    x_hbm.at[t, pl.ds(woff, 8)],
                    bufs.at[p, pl.ds(_H, 8)], sem_in),
            ]
            out = pltpu.make_async_copy(
                stage.at[p, pl.ds(0, _H)],
                out_hbm.at[t, pl.ds(off, _H)], sem_out)
            return h, merged, split, out

        def start_ins(j):
            h, merged, split, _ = unit(j)

            @pl.when(h < _NQ - 1)
            def _():
                merged.start()

            @pl.when(h == _NQ - 1)
            def _():
                for c in split:
                    c.start()

        def shift(p):
            @plsc.parallel_loop(0, _H, unroll=4)
            def _(r):
                src = bufs.at[p, r + mm]
                dst = stage.at[p, r]
                for vv in range(_NVC):
                    dst[pl.ds(vv * 16, 16)] = src[pl.ds(vv * 16, 16)]

        start_ins(0)

        def pipeline_step(j, carry):
            _, _, split_j, out_j = unit(j)
            for c in split_j:
                c.wait()

            @pl.when(j >= 1)
            def _():
                out_j.wait()

            @pl.when(j + 1 < n)
            def _():
                start_ins(j + 1)
            shift(lax.rem(j, 2))
            out_j.start()
            return carry

        lax.fori_loop(0, n, pipeline_step, 0)
        unit(n - 1)[3].wait()

    return k(xt)


def kernel(patches, m):
    del m
    xt = jnp.transpose(patches, (1, 0, 2))
    mixed = jnp.transpose(_sc_permute_t(xt), (1, 0, 2))
    ids_b = jnp.arange(_B).reshape(-1, 1)
    target = (ids_b + jnp.arange(_M)) % _B
    mix_target = (ids_b - _M + 1 + jnp.arange(_M * 2 - 1) + _B) % _B
    return (mixed, target, mix_target)
